# Initial kernel scaffold; baseline (speedup 1.0000x reference)
import jax
import jax.numpy as jnp
from jax.experimental import pallas as pl


def kernel(x, edge_index, W1, b1, W2, b2, W3, b3):
    raise NotImplementedError("write your pallas kernel here")


# speedup vs baseline: 15.8901x; 15.8901x over previous
import functools

import jax
import jax.numpy as jnp
from jax import lax
from jax.experimental import pallas as pl
from jax.experimental.pallas import tpu as pltpu
from jax.experimental.pallas import tpu_sc as plsc

N = 100000
E = 1600000
IN_DIM = 5
HIDDEN = 32
OUT_DIM = 2

HALF = 16
CW = 128
NCHUNK = 12544
EPAD = NCHUNK * CW
NPAIR = NCHUNK // 2
NPAD = 100096
ROWS_PER_TILE = NPAD // 16
NPAIR_TILE_F = NPAIR // 16
NPAIR_TILE_E = NPAIR // 32

_mesh = plsc.VectorSubcoreMesh(core_axis_name="c", subcore_axis_name="s")


def _make_agg(table_rows, feature_split):
  npair = NPAIR_TILE_F if feature_split else NPAIR_TILE_E

  @functools.partial(
      pl.kernel,
      out_type=jax.ShapeDtypeStruct((2, NPAD, HALF), jnp.float32),
      mesh=_mesh,
      compiler_params=pltpu.CompilerParams(use_tc_tiling_on_sc=False),
      scratch_types=[
          pltpu.VMEM((2, CW), jnp.int32),
          pltpu.VMEM((2, CW), jnp.int32),
          pltpu.VMEM((CW, HALF), jnp.float32),
          pltpu.VMEM((CW, HALF), jnp.float32),
          pltpu.VMEM_SHARED((NPAD, HALF), jnp.float32),
          pltpu.SemaphoreType.DMA,
          pltpu.SemaphoreType.DMA,
      ],
  )
  def agg(table_hbm, src_hbm, dst_hbm, zero_hbm, out_hbm,
          isrc, idst, rows0, rows1, acc, sem0, sem1):
    cid = lax.axis_index("c")
    sid = lax.axis_index("s")
    zlo = sid * ROWS_PER_TILE
    pltpu.sync_copy(zero_hbm.at[pl.ds(zlo, ROWS_PER_TILE)],
                    acc.at[pl.ds(zlo, ROWS_PER_TILE)])
    plsc.subcore_barrier()

    if feature_split:
      row0 = sid * npair
      splane = src_hbm.at[cid]
    else:
      row0 = cid * (NPAIR // 2) + sid * npair
      splane = src_hbm.at[0]

    @pl.loop(0, npair)
    def _(p):
      base = row0 + p
      pltpu.sync_copy(splane.at[base], isrc)
      pltpu.sync_copy(dst_hbm.at[base], idst)
      g0 = pltpu.async_copy(table_hbm.at[isrc.at[0]], rows0, sem0)
      g1 = pltpu.async_copy(table_hbm.at[isrc.at[1]], rows1, sem1)
      g0.wait()
      pltpu.sync_copy(rows0, acc.at[idst.at[0]], add=True)
      g1.wait()
      pltpu.sync_copy(rows1, acc.at[idst.at[1]], add=True)

    plsc.subcore_barrier()
    pltpu.sync_copy(acc.at[pl.ds(zlo, ROWS_PER_TILE)],
                    out_hbm.at[cid].at[pl.ds(zlo, ROWS_PER_TILE)])

  return agg


@functools.partial(
    pl.kernel,
    out_type=jax.ShapeDtypeStruct((2, NPAD, HALF), jnp.float32),
    mesh=_mesh,
    compiler_params=pltpu.CompilerParams(use_tc_tiling_on_sc=False),
    scratch_types=[
        pltpu.VMEM((2, CW), jnp.int32),
        pltpu.VMEM((CW, HALF), jnp.float32),
        pltpu.VMEM_SHARED((NPAD, HALF), jnp.float32),
    ],
)
def _degree_kernel(dst_hbm, ones_hbm, zero_hbm, out_hbm, idst, ones_v, acc):
  cid = lax.axis_index("c")
  sid = lax.axis_index("s")
  zlo = sid * ROWS_PER_TILE
  pltpu.sync_copy(zero_hbm.at[pl.ds(zlo, ROWS_PER_TILE)],
                  acc.at[pl.ds(zlo, ROWS_PER_TILE)])
  pltpu.sync_copy(ones_hbm, ones_v)
  plsc.subcore_barrier()

  row0 = cid * (NPAIR // 2) + sid * NPAIR_TILE_E

  @pl.loop(0, NPAIR_TILE_E)
  def _(p):
    pltpu.sync_copy(dst_hbm.at[row0 + p], idst)
    pltpu.sync_copy(ones_v, acc.at[idst.at[0]], add=True)
    pltpu.sync_copy(ones_v, acc.at[idst.at[1]], add=True)

  plsc.subcore_barrier()
  pltpu.sync_copy(acc.at[pl.ds(zlo, ROWS_PER_TILE)],
                  out_hbm.at[cid].at[pl.ds(zlo, ROWS_PER_TILE)])


_agg_edge_split = _make_agg(NPAD, feature_split=False)
_agg_feat_split = _make_agg(2 * NPAD, feature_split=True)


BLK = 2000
GRID = N // BLK


def _tc_prep(degp, x):
  def body(degp_ref, x_ref, dinv_ref, xp_ref):
    deg = degp_ref[0, :, 0] + degp_ref[1, :, 0] + 1.0
    dinv = lax.rsqrt(jnp.maximum(deg, 1e-12))
    dinv_ref[...] = dinv[:, None]
    xs = x_ref[...] * dinv[:, None]
    xp_ref[...] = jnp.concatenate(
        [xs, jnp.zeros((BLK, HALF - IN_DIM), jnp.float32)], axis=1)

  return pl.pallas_call(
      body,
      grid=(GRID,),
      in_specs=[
          pl.BlockSpec((2, BLK, HALF), lambda i: (0, i, 0)),
          pl.BlockSpec((BLK, IN_DIM), lambda i: (i, 0)),
      ],
      out_specs=[
          pl.BlockSpec((BLK, 1), lambda i: (i, 0)),
          pl.BlockSpec((BLK, HALF), lambda i: (i, 0)),
      ],
      out_shape=[
          jax.ShapeDtypeStruct((N, 1), jnp.float32),
          jax.ShapeDtypeStruct((NPAD, HALF), jnp.float32),
      ],
  )(degp, x)


def _tc_layer1(aggp, x, dinv, W1p, b1):
  def body(agg_ref, x_ref, dinv_ref, w_ref, b_ref, out_ref):
    dv = dinv_ref[...]
    a = agg_ref[0] + agg_ref[1]
    xpad = jnp.concatenate(
        [x_ref[...], jnp.zeros((BLK, HALF - IN_DIM), jnp.float32)], axis=1)
    s = dv * a + (dv * dv) * xpad
    h = jnp.dot(s, w_ref[...], preferred_element_type=jnp.float32)
    h = jnp.maximum(h + b_ref[...][None, :], 0.0)
    hp = dv * h
    out_ref[0] = hp[:, :HALF]
    out_ref[1] = hp[:, HALF:]

  return pl.pallas_call(
      body,
      grid=(GRID,),
      in_specs=[
          pl.BlockSpec((2, BLK, HALF), lambda i: (0, i, 0)),
          pl.BlockSpec((BLK, IN_DIM), lambda i: (i, 0)),
          pl.BlockSpec((BLK, 1), lambda i: (i, 0)),
          pl.BlockSpec((HALF, HIDDEN), lambda i: (0, 0)),
          pl.BlockSpec((HIDDEN,), lambda i: (0,)),
      ],
      out_specs=pl.BlockSpec((2, BLK, HALF), lambda i: (0, i, 0)),
      out_shape=jax.ShapeDtypeStruct((2, NPAD, HALF), jnp.float32),
  )(aggp, x, dinv, W1p, b1)


def _tc_mid(aggp, hp_prev, dinv, W, b):
  def body(agg_ref, hp_ref, dinv_ref, w_ref, b_ref, out_ref):
    dv = dinv_ref[...]
    a = jnp.concatenate([agg_ref[0], agg_ref[1]], axis=1)
    hpc = jnp.concatenate([hp_ref[0], hp_ref[1]], axis=1)
    s = dv * (a + hpc)
    h = jnp.dot(s, w_ref[...], preferred_element_type=jnp.float32)
    h = jnp.maximum(h + b_ref[...][None, :], 0.0)
    hp = dv * h
    out_ref[0] = hp[:, :HALF]
    out_ref[1] = hp[:, HALF:]

  return pl.pallas_call(
      body,
      grid=(GRID,),
      in_specs=[
          pl.BlockSpec((2, BLK, HALF), lambda i: (0, i, 0)),
          pl.BlockSpec((2, BLK, HALF), lambda i: (0, i, 0)),
          pl.BlockSpec((BLK, 1), lambda i: (i, 0)),
          pl.BlockSpec((HIDDEN, HIDDEN), lambda i: (0, 0)),
          pl.BlockSpec((HIDDEN,), lambda i: (0,)),
      ],
      out_specs=pl.BlockSpec((2, BLK, HALF), lambda i: (0, i, 0)),
      out_shape=jax.ShapeDtypeStruct((2, NPAD, HALF), jnp.float32),
  )(aggp, hp_prev, dinv, W, b)


def _tc_final(aggp, hp_prev, dinv, W3, b3):
  def body(agg_ref, hp_ref, dinv_ref, w_ref, b_ref, out_ref):
    dv = dinv_ref[...]
    a = jnp.concatenate([agg_ref[0], agg_ref[1]], axis=1)
    hpc = jnp.concatenate([hp_ref[0], hp_ref[1]], axis=1)
    s = dv * (a + hpc)
    o = jnp.dot(s, w_ref[...], preferred_element_type=jnp.float32)
    out_ref[...] = o + b_ref[...][None, :]

  return pl.pallas_call(
      body,
      grid=(GRID,),
      in_specs=[
          pl.BlockSpec((2, BLK, HALF), lambda i: (0, i, 0)),
          pl.BlockSpec((2, BLK, HALF), lambda i: (0, i, 0)),
          pl.BlockSpec((BLK, 1), lambda i: (i, 0)),
          pl.BlockSpec((HIDDEN, OUT_DIM), lambda i: (0, 0)),
          pl.BlockSpec((OUT_DIM,), lambda i: (0,)),
      ],
      out_specs=pl.BlockSpec((BLK, OUT_DIM), lambda i: (i, 0)),
      out_shape=jax.ShapeDtypeStruct((N, OUT_DIM), jnp.float32),
  )(aggp, hp_prev, dinv, W3, b3)


def kernel(x, edge_index, W1, b1, W2, b2, W3, b3):
  src = edge_index[0]
  dst = edge_index[1]
  pad = jnp.full((EPAD - E,), N, dtype=jnp.int32)
  srcp = jnp.concatenate([src, pad]).reshape(NPAIR, 2, CW)
  src2 = jnp.stack([srcp, srcp + NPAD])
  dst3 = jnp.concatenate([dst, pad]).reshape(NPAIR, 2, CW)

  zeros = jnp.zeros((NPAD, HALF), jnp.float32)
  ones128 = jnp.ones((CW, HALF), jnp.float32)
  W1p = jnp.concatenate(
      [W1, jnp.zeros((HALF - IN_DIM, HIDDEN), jnp.float32)], axis=0)

  degp = _degree_kernel(dst3, ones128, zeros)
  dinv, xp = _tc_prep(degp, x)

  agg1 = _agg_edge_split(xp, src2, dst3, zeros)
  hp1 = _tc_layer1(agg1, x, dinv, W1p, b1)

  agg2 = _agg_feat_split(hp1.reshape(2 * NPAD, HALF), src2, dst3, zeros)
  hp2 = _tc_mid(agg2, hp1, dinv, W2, b2)

  agg3 = _agg_feat_split(hp2.reshape(2 * NPAD, HALF), src2, dst3, zeros)
  return _tc_final(agg3, hp2, dinv, W3, b3)


# speedup vs baseline: 24.6775x; 1.5530x over previous
import functools

import jax
import jax.numpy as jnp
from jax import lax
from jax.experimental import pallas as pl
from jax.experimental.pallas import tpu as pltpu
from jax.experimental.pallas import tpu_sc as plsc

N = 100000
E = 1600000
IN_DIM = 5
HIDDEN = 32
OUT_DIM = 2

HALF = 16
CW = 128
NCHUNK = 12544
EPAD = NCHUNK * CW
NPAIR = NCHUNK // 2
NPAD = 100096
ROWS_PER_TILE = NPAD // 16
NPAIR_TILE_F = NPAIR // 16
NPAIR_TILE_E = NPAIR // 32

G = 2
_mesh = plsc.VectorSubcoreMesh(core_axis_name="c", subcore_axis_name="s")


def _make_agg(table_rows, feature_split):
  npair = NPAIR_TILE_F if feature_split else NPAIR_TILE_E

  @functools.partial(
      pl.kernel,
      out_type=jax.ShapeDtypeStruct((2, NPAD, HALF), jnp.float32),
      mesh=_mesh,
      compiler_params=pltpu.CompilerParams(use_tc_tiling_on_sc=False),
      scratch_types=[
          pltpu.VMEM((G, 2, CW), jnp.int32),
          pltpu.VMEM((G, 2, CW), jnp.int32),
          pltpu.VMEM((G, 2, CW), jnp.int32),
          pltpu.VMEM((G, 2, CW), jnp.int32),
          pltpu.VMEM((2 * G, CW, HALF), jnp.float32),
          pltpu.VMEM((2 * G, CW, HALF), jnp.float32),
          pltpu.VMEM_SHARED((NPAD, HALF), jnp.float32),
          pltpu.SemaphoreType.DMA,
          pltpu.SemaphoreType.DMA,
          pltpu.SemaphoreType.DMA,
          pltpu.SemaphoreType.DMA,
      ],
  )
  def agg(table_hbm, src_hbm, dst_hbm, zero_hbm, out_hbm,
          isrcA, idstA, isrcB, idstB, rowsA, rowsB, acc,
          gsemA, gsemB, ssemA, ssemB):
    cid = lax.axis_index("c")
    sid = lax.axis_index("s")
    zlo = sid * ROWS_PER_TILE
    pltpu.sync_copy(zero_hbm.at[pl.ds(zlo, ROWS_PER_TILE)],
                    acc.at[pl.ds(zlo, ROWS_PER_TILE)])
    plsc.subcore_barrier()

    if feature_split:
      row0 = sid * npair
      splane = src_hbm.at[cid]
    else:
      row0 = cid * (NPAIR // 2) + sid * npair
      splane = src_hbm.at[0]

    @pl.loop(0, npair, step=2 * G)
    def _(p):
      base = row0 + p
      pltpu.sync_copy(splane.at[pl.ds(base, G)], isrcA)
      pltpu.sync_copy(dst_hbm.at[pl.ds(base, G)], idstA)
      gA = [pltpu.async_copy(table_hbm.at[isrcA.at[j, k]],
                             rowsA.at[2 * j + k], gsemA)
            for j in range(G) for k in range(2)]
      pltpu.sync_copy(splane.at[pl.ds(base + G, G)], isrcB)
      pltpu.sync_copy(dst_hbm.at[pl.ds(base + G, G)], idstB)
      gB = [pltpu.async_copy(table_hbm.at[isrcB.at[j, k]],
                             rowsB.at[2 * j + k], gsemB)
            for j in range(G) for k in range(2)]
      for g in gA:
        g.wait()
      sA = [pltpu.async_copy(rowsA.at[2 * j + k], acc.at[idstA.at[j, k]],
                             ssemA, add=True)
            for j in range(G) for k in range(2)]
      for g in gB:
        g.wait()
      sB = [pltpu.async_copy(rowsB.at[2 * j + k], acc.at[idstB.at[j, k]],
                             ssemB, add=True)
            for j in range(G) for k in range(2)]
      for c in sA:
        c.wait()
      for c in sB:
        c.wait()

    plsc.subcore_barrier()
    pltpu.sync_copy(acc.at[pl.ds(zlo, ROWS_PER_TILE)],
                    out_hbm.at[cid].at[pl.ds(zlo, ROWS_PER_TILE)])

  return agg


@functools.partial(
    pl.kernel,
    out_type=jax.ShapeDtypeStruct((2, NPAD, HALF), jnp.float32),
    mesh=_mesh,
    compiler_params=pltpu.CompilerParams(use_tc_tiling_on_sc=False),
    scratch_types=[
        pltpu.VMEM((G, 2, CW), jnp.int32),
        pltpu.VMEM((G, 2, CW), jnp.int32),
        pltpu.VMEM((CW, HALF), jnp.float32),
        pltpu.VMEM_SHARED((NPAD, HALF), jnp.float32),
        pltpu.SemaphoreType.DMA,
        pltpu.SemaphoreType.DMA,
    ],
)
def _degree_kernel(dst_hbm, ones_hbm, zero_hbm, out_hbm,
                   idstA, idstB, ones_v, acc, semA, semB):
  cid = lax.axis_index("c")
  sid = lax.axis_index("s")
  zlo = sid * ROWS_PER_TILE
  pltpu.sync_copy(zero_hbm.at[pl.ds(zlo, ROWS_PER_TILE)],
                  acc.at[pl.ds(zlo, ROWS_PER_TILE)])
  pltpu.sync_copy(ones_hbm, ones_v)
  plsc.subcore_barrier()

  row0 = cid * (NPAIR // 2) + sid * NPAIR_TILE_E

  @pl.loop(0, NPAIR_TILE_E, step=2 * G)
  def _(p):
    base = row0 + p
    pltpu.sync_copy(dst_hbm.at[pl.ds(base, G)], idstA)
    sA = [pltpu.async_copy(ones_v, acc.at[idstA.at[j, k]], semA, add=True)
          for j in range(G) for k in range(2)]
    pltpu.sync_copy(dst_hbm.at[pl.ds(base + G, G)], idstB)
    sB = [pltpu.async_copy(ones_v, acc.at[idstB.at[j, k]], semB, add=True)
          for j in range(G) for k in range(2)]
    for c in sA:
      c.wait()
    for c in sB:
      c.wait()

  plsc.subcore_barrier()
  pltpu.sync_copy(acc.at[pl.ds(zlo, ROWS_PER_TILE)],
                  out_hbm.at[cid].at[pl.ds(zlo, ROWS_PER_TILE)])


_agg_edge_split = _make_agg(NPAD, feature_split=False)
_agg_feat_split = _make_agg(2 * NPAD, feature_split=True)


BLK = 2000
GRID = N // BLK


def _tc_prep(degp, x):
  def body(degp_ref, x_ref, dinv_ref, xp_ref):
    deg = degp_ref[0, :, 0] + degp_ref[1, :, 0] + 1.0
    dinv = lax.rsqrt(jnp.maximum(deg, 1e-12))
    dinv_ref[...] = dinv[:, None]
    xs = x_ref[...] * dinv[:, None]
    xp_ref[...] = jnp.concatenate(
        [xs, jnp.zeros((BLK, HALF - IN_DIM), jnp.float32)], axis=1)

  return pl.pallas_call(
      body,
      grid=(GRID,),
      in_specs=[
          pl.BlockSpec((2, BLK, HALF), lambda i: (0, i, 0)),
          pl.BlockSpec((BLK, IN_DIM), lambda i: (i, 0)),
      ],
      out_specs=[
          pl.BlockSpec((BLK, 1), lambda i: (i, 0)),
          pl.BlockSpec((BLK, HALF), lambda i: (i, 0)),
      ],
      out_shape=[
          jax.ShapeDtypeStruct((N, 1), jnp.float32),
          jax.ShapeDtypeStruct((NPAD, HALF), jnp.float32),
      ],
  )(degp, x)


def _tc_layer1(aggp, x, dinv, W1p, b1):
  def body(agg_ref, x_ref, dinv_ref, w_ref, b_ref, out_ref):
    dv = dinv_ref[...]
    a = agg_ref[0] + agg_ref[1]
    xpad = jnp.concatenate(
        [x_ref[...], jnp.zeros((BLK, HALF - IN_DIM), jnp.float32)], axis=1)
    s = dv * a + (dv * dv) * xpad
    h = jnp.dot(s, w_ref[...], preferred_element_type=jnp.float32)
    h = jnp.maximum(h + b_ref[...][None, :], 0.0)
    hp = dv * h
    out_ref[0] = hp[:, :HALF]
    out_ref[1] = hp[:, HALF:]

  return pl.pallas_call(
      body,
      grid=(GRID,),
      in_specs=[
          pl.BlockSpec((2, BLK, HALF), lambda i: (0, i, 0)),
          pl.BlockSpec((BLK, IN_DIM), lambda i: (i, 0)),
          pl.BlockSpec((BLK, 1), lambda i: (i, 0)),
          pl.BlockSpec((HALF, HIDDEN), lambda i: (0, 0)),
          pl.BlockSpec((HIDDEN,), lambda i: (0,)),
      ],
      out_specs=pl.BlockSpec((2, BLK, HALF), lambda i: (0, i, 0)),
      out_shape=jax.ShapeDtypeStruct((2, NPAD, HALF), jnp.float32),
  )(aggp, x, dinv, W1p, b1)


def _tc_mid(aggp, hp_prev, dinv, W, b):
  def body(agg_ref, hp_ref, dinv_ref, w_ref, b_ref, out_ref):
    dv = dinv_ref[...]
    a = jnp.concatenate([agg_ref[0], agg_ref[1]], axis=1)
    hpc = jnp.concatenate([hp_ref[0], hp_ref[1]], axis=1)
    s = dv * (a + hpc)
    h = jnp.dot(s, w_ref[...], preferred_element_type=jnp.float32)
    h = jnp.maximum(h + b_ref[...][None, :], 0.0)
    hp = dv * h
    out_ref[0] = hp[:, :HALF]
    out_ref[1] = hp[:, HALF:]

  return pl.pallas_call(
      body,
      grid=(GRID,),
      in_specs=[
          pl.BlockSpec((2, BLK, HALF), lambda i: (0, i, 0)),
          pl.BlockSpec((2, BLK, HALF), lambda i: (0, i, 0)),
          pl.BlockSpec((BLK, 1), lambda i: (i, 0)),
          pl.BlockSpec((HIDDEN, HIDDEN), lambda i: (0, 0)),
          pl.BlockSpec((HIDDEN,), lambda i: (0,)),
      ],
      out_specs=pl.BlockSpec((2, BLK, HALF), lambda i: (0, i, 0)),
      out_shape=jax.ShapeDtypeStruct((2, NPAD, HALF), jnp.float32),
  )(aggp, hp_prev, dinv, W, b)


def _tc_final(aggp, hp_prev, dinv, W3, b3):
  def body(agg_ref, hp_ref, dinv_ref, w_ref, b_ref, out_ref):
    dv = dinv_ref[...]
    a = jnp.concatenate([agg_ref[0], agg_ref[1]], axis=1)
    hpc = jnp.concatenate([hp_ref[0], hp_ref[1]], axis=1)
    s = dv * (a + hpc)
    o = jnp.dot(s, w_ref[...], preferred_element_type=jnp.float32)
    out_ref[...] = o + b_ref[...][None, :]

  return pl.pallas_call(
      body,
      grid=(GRID,),
      in_specs=[
          pl.BlockSpec((2, BLK, HALF), lambda i: (0, i, 0)),
          pl.BlockSpec((2, BLK, HALF), lambda i: (0, i, 0)),
          pl.BlockSpec((BLK, 1), lambda i: (i, 0)),
          pl.BlockSpec((HIDDEN, OUT_DIM), lambda i: (0, 0)),
          pl.BlockSpec((OUT_DIM,), lambda i: (0,)),
      ],
      out_specs=pl.BlockSpec((BLK, OUT_DIM), lambda i: (i, 0)),
      out_shape=jax.ShapeDtypeStruct((N, OUT_DIM), jnp.float32),
  )(aggp, hp_prev, dinv, W3, b3)


def kernel(x, edge_index, W1, b1, W2, b2, W3, b3):
  src = edge_index[0]
  dst = edge_index[1]
  pad = jnp.full((EPAD - E,), N, dtype=jnp.int32)
  srcp = jnp.concatenate([src, pad]).reshape(NPAIR, 2, CW)
  src2 = jnp.stack([srcp, srcp + NPAD])
  dst3 = jnp.concatenate([dst, pad]).reshape(NPAIR, 2, CW)

  zeros = jnp.zeros((NPAD, HALF), jnp.float32)
  ones128 = jnp.ones((CW, HALF), jnp.float32)
  W1p = jnp.concatenate(
      [W1, jnp.zeros((HALF - IN_DIM, HIDDEN), jnp.float32)], axis=0)

  degp = _degree_kernel(dst3, ones128, zeros)
  dinv, xp = _tc_prep(degp, x)

  agg1 = _agg_edge_split(xp, src2, dst3, zeros)
  hp1 = _tc_layer1(agg1, x, dinv, W1p, b1)

  agg2 = _agg_feat_split(hp1.reshape(2 * NPAD, HALF), src2, dst3, zeros)
  hp2 = _tc_mid(agg2, hp1, dinv, W2, b2)

  agg3 = _agg_feat_split(hp2.reshape(2 * NPAD, HALF), src2, dst3, zeros)
  return _tc_final(agg3, hp2, dinv, W3, b3)
